# baseline (speedup 1.0000x reference)
import jax
import jax.numpy as jnp
from jax.experimental import pallas as pl

G = 256
NPG = 128
INDIM = 128
EPG = 2048
D1 = 32
D2 = 32
D3 = 8
BN_EPS = 1e-5
ECH = 128
NCH = EPG // ECH

f32 = jnp.float32
bf16 = jnp.bfloat16


def _bdot(a, b, dims):
    return jax.lax.dot_general(a.astype(bf16), b.astype(bf16), (dims, ((), ())),
                               preferred_element_type=f32)


def _fdot(a, b, dims):
    return jax.lax.dot_general(a, b, (dims, ((), ())),
                               preferred_element_type=f32,
                               precision=jax.lax.Precision.HIGHEST)


def _augment(A, n):
    ii = jax.lax.broadcasted_iota(jnp.int32, (n, n), 0)
    jj = jax.lax.broadcasted_iota(jnp.int32, (n, n), 1)
    eye = (ii == jj).astype(f32)
    Ai = A + eye
    A2 = _bdot(Ai, Ai, ((1,), (0,)))
    return A2 * (1.0 - eye)


def _nnconv(xg, A2, w1, b1, Amat, Bmat, root, bias, precise_root=False):
    M = (A2 != 0.0).astype(f32)
    H = jax.nn.relu(A2 * w1 + b1) * M
    XA = _bdot(xg, Amat, ((1,), (0,)))
    XB = _bdot(xg, Bmat, ((1,), (0,)))
    aggr = _bdot(M, XB, ((0,), (0,)))
    aggr = aggr + _bdot(H, XA, ((0,), (0,)))
    dot = _fdot if precise_root else _bdot
    XR = dot(xg, root, ((1,), (0,)))
    return aggr + XR + bias


def _topk_perm(h, pw_col, n, k, precise_score=False):
    nrm = jnp.sqrt(jnp.sum(pw_col * pw_col))
    dot = _fdot if precise_score else _bdot
    sc_col = jnp.tanh(dot(h, pw_col, ((1,), (0,))) / nrm)
    sc_row = sc_col.reshape(1, n)
    ii = jax.lax.broadcasted_iota(jnp.int32, (n, n), 0)
    jj = jax.lax.broadcasted_iota(jnp.int32, (n, n), 1)
    gt = (sc_row > sc_col).astype(f32)
    eq = jnp.logical_and(sc_row == sc_col, jj < ii).astype(f32)
    rank_col = jnp.sum(gt + eq, axis=1, keepdims=True)
    rank_row = rank_col.reshape(1, n)
    rr = jax.lax.broadcasted_iota(jnp.int32, (k, n), 0).astype(f32)
    P = (rank_row == rr).astype(f32)
    sel_col = _fdot(P, sc_col, ((1,), (0,)))
    return P, sel_col


def _li_net_kernel(s_ref, d_ref, w_ref, xg_ref,
                   n1w1_ref, n1b1_ref, amat1_ref, bmat1_ref, root1_ref,
                   bias1_ref, pw1_ref,
                   n2w1_ref, n2b1_ref, amat2_ref, bmat2_ref, root2_ref,
                   bias2_ref, pw2_ref,
                   fc1_ref, fc1b_ref, bn4g_ref, bn4b_ref,
                   fc2_ref, fc2b_ref, bn5g_ref, bn5b_ref,
                   fc3_ref, fc3b_ref,
                   out_ref, sc1_ref, sc2_ref):
    s = s_ref[0]
    d = d_ref[0]
    w = w_ref[0]
    xg = xg_ref[0]

    iota_n = jax.lax.broadcasted_iota(jnp.int32, (NPG, ECH), 0)
    A = jnp.zeros((NPG, NPG), f32)
    for c in range(NCH):
        srow = s[c:c + 1, :]
        drow = d[c:c + 1, :]
        wrow = w[c:c + 1, :]
        sT = (iota_n == srow).astype(f32)
        dwT = (iota_n == drow).astype(f32) * wrow
        A = A + _fdot(sT, dwT, ((1,), (1,)))

    A2 = _augment(A, NPG)
    h1 = _nnconv(xg, A2, n1w1_ref[:], n1b1_ref[:], amat1_ref[:], bmat1_ref[:],
                 root1_ref[:], bias1_ref[:])
    k1 = NPG // 2
    P1, sel1 = _topk_perm(h1, pw1_ref[:], NPG, k1)
    xk1 = _fdot(P1, h1, ((1,), (0,))) * sel1
    AP = _fdot(A2, P1, ((1,), (1,)))
    A1p = _fdot(P1, AP, ((1,), (0,)))
    x1 = jnp.concatenate([jnp.max(xk1, axis=0, keepdims=True),
                          jnp.mean(xk1, axis=0, keepdims=True)], axis=1)

    A2b = _augment(A1p, k1)
    h2 = _nnconv(xk1, A2b, n2w1_ref[:], n2b1_ref[:], amat2_ref[:],
                 bmat2_ref[:], root2_ref[:], bias2_ref[:], precise_root=True)
    k2 = k1 // 2
    P2, sel2 = _topk_perm(h2, pw2_ref[:], k1, k2, precise_score=True)
    xk2 = _fdot(P2, h2, ((1,), (0,))) * sel2
    x2 = jnp.concatenate([jnp.max(xk2, axis=0, keepdims=True),
                          jnp.mean(xk2, axis=0, keepdims=True)], axis=1)

    xcat = jnp.concatenate([x1, x2], axis=1)
    rsq = jnp.sqrt(1.0 + BN_EPS)
    h = _bdot(xcat, fc1_ref[:], ((1,), (0,))) + fc1b_ref[:]
    h = jax.nn.relu(bn4g_ref[:] * h / rsq + bn4b_ref[:])
    h = _bdot(h, fc2_ref[:], ((1,), (0,))) + fc2b_ref[:]
    h = jax.nn.relu(bn5g_ref[:] * h / rsq + bn5b_ref[:])
    lg = _bdot(h, fc3_ref[:], ((1,), (0,))) + fc3b_ref[:]
    m = jnp.max(lg, axis=1, keepdims=True)
    lse = m + jnp.log(jnp.sum(jnp.exp(lg - m), axis=1, keepdims=True))
    out_ref[0] = lg - lse
    sc1_ref[0] = sel1.reshape(1, k1)
    sc2_ref[0] = sel2.reshape(1, k2)


def kernel(x, edge_index, batch, edge_attr, n1_w1, n1_b1, n1_w2, n1_b2,
           conv1_root, conv1_bias, pool1_w, n2_w1, n2_b1, n2_w2, n2_b2,
           conv2_root, conv2_bias, pool2_w, fc1_w, fc1_b, bn4_g, bn4_b,
           fc2_w, fc2_b, bn5_g, bn5_b, fc3_w, fc3_b):
    k1 = NPG // 2
    k2 = k1 // 2
    s = (edge_index[0] % NPG).astype(jnp.int32).reshape(G, NCH, ECH)
    d = (edge_index[1] % NPG).astype(jnp.int32).reshape(G, NCH, ECH)
    w = edge_attr.reshape(G, NCH, ECH).astype(f32)
    xg = x.reshape(G, NPG, INDIM)

    n1w1 = n1_w1.reshape(1, 1)
    n1b1 = n1_b1.reshape(1, 1)
    amat1 = n1_w2.reshape(INDIM, D1)
    bmat1 = n1_b2.reshape(INDIM, D1)
    bias1 = conv1_bias.reshape(1, D1)
    pw1 = pool1_w.reshape(D1, 1)
    n2w1 = n2_w1.reshape(1, 1)
    n2b1 = n2_b1.reshape(1, 1)
    amat2 = n2_w2.reshape(D1, D2)
    bmat2 = n2_b2.reshape(D1, D2)
    bias2 = conv2_bias.reshape(1, D2)
    pw2 = pool2_w.reshape(D2, 1)
    fc1t = fc1_w.T
    fc2t = fc2_w.T
    fc3t = fc3_w.T
    fc1b = fc1_b.reshape(1, D2)
    fc2b = fc2_b.reshape(1, D3)
    fc3b = fc3_b.reshape(1, 2)
    bn4g = bn4_g.reshape(1, D2)
    bn4b = bn4_b.reshape(1, D2)
    bn5g = bn5_g.reshape(1, D3)
    bn5b = bn5_b.reshape(1, D3)

    def full(a):
        return pl.BlockSpec(a.shape, lambda i: (0,) * a.ndim)

    grid = (G,)
    in_specs = [
        pl.BlockSpec((1, NCH, ECH), lambda i: (i, 0, 0)),
        pl.BlockSpec((1, NCH, ECH), lambda i: (i, 0, 0)),
        pl.BlockSpec((1, NCH, ECH), lambda i: (i, 0, 0)),
        pl.BlockSpec((1, NPG, INDIM), lambda i: (i, 0, 0)),
    ] + [full(a) for a in (n1w1, n1b1, amat1, bmat1, conv1_root, bias1, pw1,
                           n2w1, n2b1, amat2, bmat2, conv2_root, bias2, pw2,
                           fc1t, fc1b, bn4g, bn4b, fc2t, fc2b, bn5g, bn5b,
                           fc3t, fc3b)]
    out_specs = [
        pl.BlockSpec((1, 1, 2), lambda i: (i, 0, 0)),
        pl.BlockSpec((1, 1, k1), lambda i: (i, 0, 0)),
        pl.BlockSpec((1, 1, k2), lambda i: (i, 0, 0)),
    ]
    out_shape = [
        jax.ShapeDtypeStruct((G, 1, 2), f32),
        jax.ShapeDtypeStruct((G, 1, k1), f32),
        jax.ShapeDtypeStruct((G, 1, k2), f32),
    ]
    out, sc1, sc2 = pl.pallas_call(
        _li_net_kernel,
        grid=grid,
        in_specs=in_specs,
        out_specs=out_specs,
        out_shape=out_shape,
    )(s, d, w, xg, n1w1, n1b1, amat1, bmat1, conv1_root, bias1, pw1,
      n2w1, n2b1, amat2, bmat2, conv2_root, bias2, pw2,
      fc1t, fc1b, bn4g, bn4b, fc2t, fc2b, bn5g, bn5b, fc3t, fc3b)
    return (out.reshape(G, 2), sc1.reshape(-1), sc2.reshape(-1))


# speedup vs baseline: 1.0580x; 1.0580x over previous
import jax
import jax.numpy as jnp
from jax.experimental import pallas as pl

G = 256
NPG = 128
INDIM = 128
EPG = 2048
D1 = 32
D2 = 32
D3 = 8
BN_EPS = 1e-5
ECH = 128
NCH = EPG // ECH

f32 = jnp.float32
bf16 = jnp.bfloat16


def _bdot(a, b, dims):
    return jax.lax.dot_general(a.astype(bf16), b.astype(bf16), (dims, ((), ())),
                               preferred_element_type=f32)


def _fdot(a, b, dims):
    return jax.lax.dot_general(a, b, (dims, ((), ())),
                               preferred_element_type=f32,
                               precision=jax.lax.Precision.HIGHEST)


def _augment(A, n):
    ii = jax.lax.broadcasted_iota(jnp.int32, (n, n), 0)
    jj = jax.lax.broadcasted_iota(jnp.int32, (n, n), 1)
    eye = (ii == jj).astype(f32)
    Ai = A + eye
    A2 = _bdot(Ai, Ai, ((1,), (0,)))
    return A2 * (1.0 - eye)


def _nnconv(xg, A2, w1, b1, Amat, Bmat, root, bias, precise_root=False):
    M = (A2 != 0.0).astype(f32)
    H = jax.nn.relu(A2 * w1 + b1) * M
    XA = _bdot(xg, Amat, ((1,), (0,)))
    XB = _bdot(xg, Bmat, ((1,), (0,)))
    aggr = _bdot(M, XB, ((0,), (0,)))
    aggr = aggr + _bdot(H, XA, ((0,), (0,)))
    dot = _fdot if precise_root else _bdot
    XR = dot(xg, root, ((1,), (0,)))
    return aggr + XR + bias


def _topk_perm(h, pw_col, n, k, precise_score=False):
    nrm = jnp.sqrt(jnp.sum(pw_col * pw_col))
    dot = _fdot if precise_score else _bdot
    sc_col = jnp.tanh(dot(h, pw_col, ((1,), (0,))) / nrm)
    sc_row = sc_col.reshape(1, n)
    ii = jax.lax.broadcasted_iota(jnp.int32, (n, n), 0)
    jj = jax.lax.broadcasted_iota(jnp.int32, (n, n), 1)
    gt = (sc_row > sc_col).astype(f32)
    eq = jnp.logical_and(sc_row == sc_col, jj < ii).astype(f32)
    rank_col = jnp.sum(gt + eq, axis=1, keepdims=True)
    rank_row = rank_col.reshape(1, n)
    rr = jax.lax.broadcasted_iota(jnp.int32, (k, n), 0).astype(f32)
    P = (rank_row == rr).astype(f32)
    sel_col = _fdot(P, sc_col, ((1,), (0,)))
    return P, sel_col


def _li_net_kernel(s_ref, d_ref, w_ref, xg_ref,
                   n1w1_ref, n1b1_ref, amat1_ref, bmat1_ref, root1_ref,
                   bias1_ref, pw1_ref,
                   n2w1_ref, n2b1_ref, amat2_ref, bmat2_ref, root2_ref,
                   bias2_ref, pw2_ref,
                   fc1_ref, fc1b_ref, bn4g_ref, bn4b_ref,
                   fc2_ref, fc2b_ref, bn5g_ref, bn5b_ref,
                   fc3_ref, fc3b_ref,
                   out_ref, sc1_ref, sc2_ref):
    srow = s_ref[0]
    drow = d_ref[0]
    wrow = w_ref[0]
    xg = xg_ref[0]

    iota_n = jax.lax.broadcasted_iota(jnp.int32, (NPG, EPG), 0)
    sT = (iota_n == srow).astype(f32)
    dwT = (iota_n == drow).astype(f32) * wrow
    A = _fdot(sT, dwT, ((1,), (1,)))

    A2 = _augment(A, NPG)
    h1 = _nnconv(xg, A2, n1w1_ref[:], n1b1_ref[:], amat1_ref[:], bmat1_ref[:],
                 root1_ref[:], bias1_ref[:])
    k1 = NPG // 2
    P1, sel1 = _topk_perm(h1, pw1_ref[:], NPG, k1)
    xk1 = _fdot(P1, h1, ((1,), (0,))) * sel1
    AP = _fdot(A2, P1, ((1,), (1,)))
    A1p = _fdot(P1, AP, ((1,), (0,)))
    x1 = jnp.concatenate([jnp.max(xk1, axis=0, keepdims=True),
                          jnp.mean(xk1, axis=0, keepdims=True)], axis=1)

    A2b = _augment(A1p, k1)
    h2 = _nnconv(xk1, A2b, n2w1_ref[:], n2b1_ref[:], amat2_ref[:],
                 bmat2_ref[:], root2_ref[:], bias2_ref[:], precise_root=True)
    k2 = k1 // 2
    P2, sel2 = _topk_perm(h2, pw2_ref[:], k1, k2, precise_score=True)
    xk2 = _fdot(P2, h2, ((1,), (0,))) * sel2
    x2 = jnp.concatenate([jnp.max(xk2, axis=0, keepdims=True),
                          jnp.mean(xk2, axis=0, keepdims=True)], axis=1)

    xcat = jnp.concatenate([x1, x2], axis=1)
    rsq = jnp.sqrt(1.0 + BN_EPS)
    h = _bdot(xcat, fc1_ref[:], ((1,), (0,))) + fc1b_ref[:]
    h = jax.nn.relu(bn4g_ref[:] * h / rsq + bn4b_ref[:])
    h = _bdot(h, fc2_ref[:], ((1,), (0,))) + fc2b_ref[:]
    h = jax.nn.relu(bn5g_ref[:] * h / rsq + bn5b_ref[:])
    lg = _bdot(h, fc3_ref[:], ((1,), (0,))) + fc3b_ref[:]
    m = jnp.max(lg, axis=1, keepdims=True)
    lse = m + jnp.log(jnp.sum(jnp.exp(lg - m), axis=1, keepdims=True))
    out_ref[0] = lg - lse
    sc1_ref[0] = sel1.reshape(1, k1)
    sc2_ref[0] = sel2.reshape(1, k2)


def kernel(x, edge_index, batch, edge_attr, n1_w1, n1_b1, n1_w2, n1_b2,
           conv1_root, conv1_bias, pool1_w, n2_w1, n2_b1, n2_w2, n2_b2,
           conv2_root, conv2_bias, pool2_w, fc1_w, fc1_b, bn4_g, bn4_b,
           fc2_w, fc2_b, bn5_g, bn5_b, fc3_w, fc3_b):
    k1 = NPG // 2
    k2 = k1 // 2
    s = (edge_index[0] % NPG).astype(jnp.int32).reshape(G, 1, EPG)
    d = (edge_index[1] % NPG).astype(jnp.int32).reshape(G, 1, EPG)
    w = edge_attr.reshape(G, 1, EPG).astype(f32)
    xg = x.reshape(G, NPG, INDIM)

    n1w1 = n1_w1.reshape(1, 1)
    n1b1 = n1_b1.reshape(1, 1)
    amat1 = n1_w2.reshape(INDIM, D1)
    bmat1 = n1_b2.reshape(INDIM, D1)
    bias1 = conv1_bias.reshape(1, D1)
    pw1 = pool1_w.reshape(D1, 1)
    n2w1 = n2_w1.reshape(1, 1)
    n2b1 = n2_b1.reshape(1, 1)
    amat2 = n2_w2.reshape(D1, D2)
    bmat2 = n2_b2.reshape(D1, D2)
    bias2 = conv2_bias.reshape(1, D2)
    pw2 = pool2_w.reshape(D2, 1)
    fc1t = fc1_w.T
    fc2t = fc2_w.T
    fc3t = fc3_w.T
    fc1b = fc1_b.reshape(1, D2)
    fc2b = fc2_b.reshape(1, D3)
    fc3b = fc3_b.reshape(1, 2)
    bn4g = bn4_g.reshape(1, D2)
    bn4b = bn4_b.reshape(1, D2)
    bn5g = bn5_g.reshape(1, D3)
    bn5b = bn5_b.reshape(1, D3)

    def full(a):
        return pl.BlockSpec(a.shape, lambda i: (0,) * a.ndim)

    grid = (G,)
    in_specs = [
        pl.BlockSpec((1, 1, EPG), lambda i: (i, 0, 0)),
        pl.BlockSpec((1, 1, EPG), lambda i: (i, 0, 0)),
        pl.BlockSpec((1, 1, EPG), lambda i: (i, 0, 0)),
        pl.BlockSpec((1, NPG, INDIM), lambda i: (i, 0, 0)),
    ] + [full(a) for a in (n1w1, n1b1, amat1, bmat1, conv1_root, bias1, pw1,
                           n2w1, n2b1, amat2, bmat2, conv2_root, bias2, pw2,
                           fc1t, fc1b, bn4g, bn4b, fc2t, fc2b, bn5g, bn5b,
                           fc3t, fc3b)]
    out_specs = [
        pl.BlockSpec((1, 1, 2), lambda i: (i, 0, 0)),
        pl.BlockSpec((1, 1, k1), lambda i: (i, 0, 0)),
        pl.BlockSpec((1, 1, k2), lambda i: (i, 0, 0)),
    ]
    out_shape = [
        jax.ShapeDtypeStruct((G, 1, 2), f32),
        jax.ShapeDtypeStruct((G, 1, k1), f32),
        jax.ShapeDtypeStruct((G, 1, k2), f32),
    ]
    out, sc1, sc2 = pl.pallas_call(
        _li_net_kernel,
        grid=grid,
        in_specs=in_specs,
        out_specs=out_specs,
        out_shape=out_shape,
    )(s, d, w, xg, n1w1, n1b1, amat1, bmat1, conv1_root, bias1, pw1,
      n2w1, n2b1, amat2, bmat2, conv2_root, bias2, pw2,
      fc1t, fc1b, bn4g, bn4b, fc2t, fc2b, bn5g, bn5b, fc3t, fc3b)
    return (out.reshape(G, 2), sc1.reshape(-1), sc2.reshape(-1))


# speedup vs baseline: 2.0740x; 1.9602x over previous
import jax
import jax.numpy as jnp
from jax.experimental import pallas as pl

G = 256
NPG = 128
INDIM = 128
EPG = 2048
D1 = 32
D2 = 32
D3 = 8
BN_EPS = 1e-5
ECH = 128
NCH = EPG // ECH

f32 = jnp.float32
bf16 = jnp.bfloat16


def _bdot(a, b, dims):
    return jax.lax.dot_general(a.astype(bf16), b.astype(bf16), (dims, ((), ())),
                               preferred_element_type=f32)


def _fdot(a, b, dims):
    return jax.lax.dot_general(a, b, (dims, ((), ())),
                               preferred_element_type=f32,
                               precision=jax.lax.Precision.HIGHEST)


def _augment(A, n):
    ii = jax.lax.broadcasted_iota(jnp.int32, (n, n), 0)
    jj = jax.lax.broadcasted_iota(jnp.int32, (n, n), 1)
    eye = (ii == jj).astype(f32)
    Ai = A + eye
    A2 = _bdot(Ai, Ai, ((1,), (0,)))
    return A2 * (1.0 - eye)


def _nnconv(xg, A2, w1, b1, Amat, Bmat, root, bias, precise_root=False):
    M = (A2 != 0.0).astype(f32)
    H = jax.nn.relu(A2 * w1 + b1) * M
    XA = _bdot(xg, Amat, ((1,), (0,)))
    XB = _bdot(xg, Bmat, ((1,), (0,)))
    aggr = _bdot(M, XB, ((0,), (0,)))
    aggr = aggr + _bdot(H, XA, ((0,), (0,)))
    dot = _fdot if precise_root else _bdot
    XR = dot(xg, root, ((1,), (0,)))
    return aggr + XR + bias


def _topk_perm(h, pw_col, n, k, precise_score=False):
    nrm = jnp.sqrt(jnp.sum(pw_col * pw_col))
    dot = _fdot if precise_score else _bdot
    sc_col = jnp.tanh(dot(h, pw_col, ((1,), (0,))) / nrm)
    sc_row = sc_col.reshape(1, n)
    ii = jax.lax.broadcasted_iota(jnp.int32, (n, n), 0)
    jj = jax.lax.broadcasted_iota(jnp.int32, (n, n), 1)
    gt = sc_col > sc_row
    eq = jnp.logical_and(sc_col == sc_row, ii < jj)
    Gm = jnp.logical_or(gt, eq).astype(bf16)
    rank_row = _bdot(jnp.ones((1, n), f32), Gm, ((1,), (0,)))
    rr = jax.lax.broadcasted_iota(jnp.int32, (k, n), 0).astype(f32)
    P = (rank_row == rr).astype(f32)
    sel_col = _fdot(P, sc_col, ((1,), (0,)))
    return P, sel_col


BPP = 2


def _li_net_kernel(s_ref, d_ref, w_ref, xg_ref,
                   n1w1_ref, n1b1_ref, amat1_ref, bmat1_ref, root1_ref,
                   bias1_ref, pw1_ref,
                   n2w1_ref, n2b1_ref, amat2_ref, bmat2_ref, root2_ref,
                   bias2_ref, pw2_ref,
                   fc1_ref, fc1b_ref, bn4g_ref, bn4b_ref,
                   fc2_ref, fc2b_ref, bn5g_ref, bn5b_ref,
                   fc3_ref, fc3b_ref,
                   out_ref, sc1_ref, sc2_ref):
    k1 = NPG // 2
    k2 = k1 // 2
    for g in range(BPP):
        srow = s_ref[g]
        drow = d_ref[g]
        wrow = w_ref[g]
        xg = xg_ref[g]

        iota_n = jax.lax.broadcasted_iota(jnp.int32, (NPG, EPG), 0)
        sT = (iota_n == srow).astype(f32)
        dwT = (iota_n == drow).astype(f32) * wrow
        A = _fdot(sT, dwT, ((1,), (1,)))

        A2 = _augment(A, NPG)
        h1 = _nnconv(xg, A2, n1w1_ref[:], n1b1_ref[:], amat1_ref[:],
                     bmat1_ref[:], root1_ref[:], bias1_ref[:])
        P1, sel1 = _topk_perm(h1, pw1_ref[:], NPG, k1)
        xk1 = _fdot(P1, h1, ((1,), (0,))) * sel1
        AP = _fdot(A2, P1, ((1,), (1,)))
        A1p = _fdot(P1, AP, ((1,), (0,)))
        x1 = jnp.concatenate([jnp.max(xk1, axis=0, keepdims=True),
                              jnp.mean(xk1, axis=0, keepdims=True)], axis=1)

        A2b = _augment(A1p, k1)
        h2 = _nnconv(xk1, A2b, n2w1_ref[:], n2b1_ref[:], amat2_ref[:],
                     bmat2_ref[:], root2_ref[:], bias2_ref[:],
                     precise_root=True)
        P2, sel2 = _topk_perm(h2, pw2_ref[:], k1, k2, precise_score=True)
        xk2 = _fdot(P2, h2, ((1,), (0,))) * sel2
        x2 = jnp.concatenate([jnp.max(xk2, axis=0, keepdims=True),
                              jnp.mean(xk2, axis=0, keepdims=True)], axis=1)

        xcat = jnp.concatenate([x1, x2], axis=1)
        rsq = jnp.sqrt(1.0 + BN_EPS)
        h = _bdot(xcat, fc1_ref[:], ((1,), (0,))) + fc1b_ref[:]
        h = jax.nn.relu(bn4g_ref[:] * h / rsq + bn4b_ref[:])
        h = _bdot(h, fc2_ref[:], ((1,), (0,))) + fc2b_ref[:]
        h = jax.nn.relu(bn5g_ref[:] * h / rsq + bn5b_ref[:])
        lg = _bdot(h, fc3_ref[:], ((1,), (0,))) + fc3b_ref[:]
        m = jnp.max(lg, axis=1, keepdims=True)
        lse = m + jnp.log(jnp.sum(jnp.exp(lg - m), axis=1, keepdims=True))
        out_ref[g] = lg - lse
        sc1_ref[g] = sel1.reshape(1, k1)
        sc2_ref[g] = sel2.reshape(1, k2)


def kernel(x, edge_index, batch, edge_attr, n1_w1, n1_b1, n1_w2, n1_b2,
           conv1_root, conv1_bias, pool1_w, n2_w1, n2_b1, n2_w2, n2_b2,
           conv2_root, conv2_bias, pool2_w, fc1_w, fc1_b, bn4_g, bn4_b,
           fc2_w, fc2_b, bn5_g, bn5_b, fc3_w, fc3_b):
    k1 = NPG // 2
    k2 = k1 // 2
    s = (edge_index[0] % NPG).astype(jnp.int32).reshape(G, 1, EPG)
    d = (edge_index[1] % NPG).astype(jnp.int32).reshape(G, 1, EPG)
    w = edge_attr.reshape(G, 1, EPG).astype(f32)
    xg = x.reshape(G, NPG, INDIM)

    n1w1 = n1_w1.reshape(1, 1)
    n1b1 = n1_b1.reshape(1, 1)
    amat1 = n1_w2.reshape(INDIM, D1)
    bmat1 = n1_b2.reshape(INDIM, D1)
    bias1 = conv1_bias.reshape(1, D1)
    pw1 = pool1_w.reshape(D1, 1)
    n2w1 = n2_w1.reshape(1, 1)
    n2b1 = n2_b1.reshape(1, 1)
    amat2 = n2_w2.reshape(D1, D2)
    bmat2 = n2_b2.reshape(D1, D2)
    bias2 = conv2_bias.reshape(1, D2)
    pw2 = pool2_w.reshape(D2, 1)
    fc1t = fc1_w.T
    fc2t = fc2_w.T
    fc3t = fc3_w.T
    fc1b = fc1_b.reshape(1, D2)
    fc2b = fc2_b.reshape(1, D3)
    fc3b = fc3_b.reshape(1, 2)
    bn4g = bn4_g.reshape(1, D2)
    bn4b = bn4_b.reshape(1, D2)
    bn5g = bn5_g.reshape(1, D3)
    bn5b = bn5_b.reshape(1, D3)

    def full(a):
        return pl.BlockSpec(a.shape, lambda i: (0,) * a.ndim)

    grid = (G // BPP,)
    in_specs = [
        pl.BlockSpec((BPP, 1, EPG), lambda i: (i, 0, 0)),
        pl.BlockSpec((BPP, 1, EPG), lambda i: (i, 0, 0)),
        pl.BlockSpec((BPP, 1, EPG), lambda i: (i, 0, 0)),
        pl.BlockSpec((BPP, NPG, INDIM), lambda i: (i, 0, 0)),
    ] + [full(a) for a in (n1w1, n1b1, amat1, bmat1, conv1_root, bias1, pw1,
                           n2w1, n2b1, amat2, bmat2, conv2_root, bias2, pw2,
                           fc1t, fc1b, bn4g, bn4b, fc2t, fc2b, bn5g, bn5b,
                           fc3t, fc3b)]
    out_specs = [
        pl.BlockSpec((BPP, 1, 2), lambda i: (i, 0, 0)),
        pl.BlockSpec((BPP, 1, k1), lambda i: (i, 0, 0)),
        pl.BlockSpec((BPP, 1, k2), lambda i: (i, 0, 0)),
    ]
    out_shape = [
        jax.ShapeDtypeStruct((G, 1, 2), f32),
        jax.ShapeDtypeStruct((G, 1, k1), f32),
        jax.ShapeDtypeStruct((G, 1, k2), f32),
    ]
    out, sc1, sc2 = pl.pallas_call(
        _li_net_kernel,
        grid=grid,
        in_specs=in_specs,
        out_specs=out_specs,
        out_shape=out_shape,
    )(s, d, w, xg, n1w1, n1b1, amat1, bmat1, conv1_root, bias1, pw1,
      n2w1, n2b1, amat2, bmat2, conv2_root, bias2, pw2,
      fc1t, fc1b, bn4g, bn4b, fc2t, fc2b, bn5g, bn5b, fc3t, fc3b)
    return (out.reshape(G, 2), sc1.reshape(-1), sc2.reshape(-1))


# speedup vs baseline: 2.9135x; 1.4048x over previous
import jax
import jax.numpy as jnp
from jax.experimental import pallas as pl

G = 256
NPG = 128
INDIM = 128
EPG = 2048
D1 = 32
D2 = 32
D3 = 8
BN_EPS = 1e-5
ECH = 128
NCH = EPG // ECH

f32 = jnp.float32
bf16 = jnp.bfloat16


def _bdot(a, b, dims):
    return jax.lax.dot_general(a.astype(bf16), b.astype(bf16), (dims, ((), ())),
                               preferred_element_type=f32)


def _fdot(a, b, dims):
    return jax.lax.dot_general(a, b, (dims, ((), ())),
                               preferred_element_type=f32,
                               precision=jax.lax.Precision.HIGHEST)


def _augment(A, n):
    ii = jax.lax.broadcasted_iota(jnp.int32, (n, n), 0)
    jj = jax.lax.broadcasted_iota(jnp.int32, (n, n), 1)
    eye = (ii == jj).astype(f32)
    Ai = A + eye
    A2 = _bdot(Ai, Ai, ((1,), (0,)))
    return A2 * (1.0 - eye)


def _nnconv(xg, A2, w1, b1, Amat, Bmat, root, bias, precise_root=False):
    M = (A2 != 0.0).astype(f32)
    H = jax.nn.relu(A2 * w1 + b1) * M
    XA = _bdot(xg, Amat, ((1,), (0,)))
    XB = _bdot(xg, Bmat, ((1,), (0,)))
    aggr = _bdot(M, XB, ((0,), (0,)))
    aggr = aggr + _bdot(H, XA, ((0,), (0,)))
    dot = _fdot if precise_root else _bdot
    XR = dot(xg, root, ((1,), (0,)))
    return aggr + XR + bias


def _topk_perm(h, pw_col, n, k, precise_score=False):
    nrm = jnp.sqrt(jnp.sum(pw_col * pw_col))
    dot = _fdot if precise_score else _bdot
    sc_col = jnp.tanh(dot(h, pw_col, ((1,), (0,))) / nrm)
    sc_row = sc_col.reshape(1, n)
    ii = jax.lax.broadcasted_iota(jnp.int32, (n, n), 0)
    jj = jax.lax.broadcasted_iota(jnp.int32, (n, n), 1)
    gt = sc_col > sc_row
    eq = jnp.logical_and(sc_col == sc_row, ii < jj)
    Gm = jnp.logical_or(gt, eq).astype(bf16)
    rank_row = _bdot(jnp.ones((1, n), f32), Gm, ((1,), (0,)))
    rr = jax.lax.broadcasted_iota(jnp.int32, (k, n), 0).astype(f32)
    P = (rank_row == rr).astype(f32)
    sel_col = _fdot(P, sc_col, ((1,), (0,)))
    return P, sel_col


BPP = 4


def _li_net_kernel(s_ref, d_ref, w_ref, xg_ref,
                   n1w1_ref, n1b1_ref, amat1_ref, bmat1_ref, root1_ref,
                   bias1_ref, pw1_ref,
                   n2w1_ref, n2b1_ref, amat2_ref, bmat2_ref, root2_ref,
                   bias2_ref, pw2_ref,
                   fc1_ref, fc1b_ref, bn4g_ref, bn4b_ref,
                   fc2_ref, fc2b_ref, bn5g_ref, bn5b_ref,
                   fc3_ref, fc3b_ref,
                   out_ref, sc1_ref, sc2_ref):
    k1 = NPG // 2
    k2 = k1 // 2
    for g in range(BPP):
        srow = s_ref[g]
        drow = d_ref[g]
        wrow = w_ref[g]
        xg = xg_ref[g]

        iota_n = jax.lax.broadcasted_iota(jnp.int32, (NPG, EPG), 0)
        sT = (iota_n == srow).astype(f32)
        dT = (iota_n == drow).astype(f32)
        sTb = sT.astype(bf16)
        q0 = wrow.astype(bf16).astype(f32)
        q1 = (wrow - q0).astype(bf16).astype(f32)
        q2 = wrow - q0 - q1
        A = jnp.zeros((NPG, NPG), f32)
        for piece in (q0, q1, q2):
            dw = (dT * piece).astype(bf16)
            A = A + jax.lax.dot_general(sTb, dw, ((((1,), (1,))), ((), ())),
                                        preferred_element_type=f32)

        A2 = _augment(A, NPG)
        h1 = _nnconv(xg, A2, n1w1_ref[:], n1b1_ref[:], amat1_ref[:],
                     bmat1_ref[:], root1_ref[:], bias1_ref[:])
        P1, sel1 = _topk_perm(h1, pw1_ref[:], NPG, k1)
        xk1 = _fdot(P1, h1, ((1,), (0,))) * sel1
        AP = _fdot(A2, P1, ((1,), (1,)))
        A1p = _fdot(P1, AP, ((1,), (0,)))
        x1 = jnp.concatenate([jnp.max(xk1, axis=0, keepdims=True),
                              jnp.mean(xk1, axis=0, keepdims=True)], axis=1)

        A2b = _augment(A1p, k1)
        h2 = _nnconv(xk1, A2b, n2w1_ref[:], n2b1_ref[:], amat2_ref[:],
                     bmat2_ref[:], root2_ref[:], bias2_ref[:],
                     precise_root=True)
        P2, sel2 = _topk_perm(h2, pw2_ref[:], k1, k2, precise_score=True)
        xk2 = _fdot(P2, h2, ((1,), (0,))) * sel2
        x2 = jnp.concatenate([jnp.max(xk2, axis=0, keepdims=True),
                              jnp.mean(xk2, axis=0, keepdims=True)], axis=1)

        xcat = jnp.concatenate([x1, x2], axis=1)
        rsq = jnp.sqrt(1.0 + BN_EPS)
        h = _bdot(xcat, fc1_ref[:], ((1,), (0,))) + fc1b_ref[:]
        h = jax.nn.relu(bn4g_ref[:] * h / rsq + bn4b_ref[:])
        h = _bdot(h, fc2_ref[:], ((1,), (0,))) + fc2b_ref[:]
        h = jax.nn.relu(bn5g_ref[:] * h / rsq + bn5b_ref[:])
        lg = _bdot(h, fc3_ref[:], ((1,), (0,))) + fc3b_ref[:]
        m = jnp.max(lg, axis=1, keepdims=True)
        lse = m + jnp.log(jnp.sum(jnp.exp(lg - m), axis=1, keepdims=True))
        out_ref[g] = lg - lse
        sc1_ref[g] = sel1.reshape(1, k1)
        sc2_ref[g] = sel2.reshape(1, k2)


def kernel(x, edge_index, batch, edge_attr, n1_w1, n1_b1, n1_w2, n1_b2,
           conv1_root, conv1_bias, pool1_w, n2_w1, n2_b1, n2_w2, n2_b2,
           conv2_root, conv2_bias, pool2_w, fc1_w, fc1_b, bn4_g, bn4_b,
           fc2_w, fc2_b, bn5_g, bn5_b, fc3_w, fc3_b):
    k1 = NPG // 2
    k2 = k1 // 2
    s = (edge_index[0] % NPG).astype(jnp.int32).reshape(G, 1, EPG)
    d = (edge_index[1] % NPG).astype(jnp.int32).reshape(G, 1, EPG)
    w = edge_attr.reshape(G, 1, EPG).astype(f32)
    xg = x.reshape(G, NPG, INDIM)

    n1w1 = n1_w1.reshape(1, 1)
    n1b1 = n1_b1.reshape(1, 1)
    amat1 = n1_w2.reshape(INDIM, D1)
    bmat1 = n1_b2.reshape(INDIM, D1)
    bias1 = conv1_bias.reshape(1, D1)
    pw1 = pool1_w.reshape(D1, 1)
    n2w1 = n2_w1.reshape(1, 1)
    n2b1 = n2_b1.reshape(1, 1)
    amat2 = n2_w2.reshape(D1, D2)
    bmat2 = n2_b2.reshape(D1, D2)
    bias2 = conv2_bias.reshape(1, D2)
    pw2 = pool2_w.reshape(D2, 1)
    fc1t = fc1_w.T
    fc2t = fc2_w.T
    fc3t = fc3_w.T
    fc1b = fc1_b.reshape(1, D2)
    fc2b = fc2_b.reshape(1, D3)
    fc3b = fc3_b.reshape(1, 2)
    bn4g = bn4_g.reshape(1, D2)
    bn4b = bn4_b.reshape(1, D2)
    bn5g = bn5_g.reshape(1, D3)
    bn5b = bn5_b.reshape(1, D3)

    def full(a):
        return pl.BlockSpec(a.shape, lambda i: (0,) * a.ndim)

    grid = (G // BPP,)
    in_specs = [
        pl.BlockSpec((BPP, 1, EPG), lambda i: (i, 0, 0)),
        pl.BlockSpec((BPP, 1, EPG), lambda i: (i, 0, 0)),
        pl.BlockSpec((BPP, 1, EPG), lambda i: (i, 0, 0)),
        pl.BlockSpec((BPP, NPG, INDIM), lambda i: (i, 0, 0)),
    ] + [full(a) for a in (n1w1, n1b1, amat1, bmat1, conv1_root, bias1, pw1,
                           n2w1, n2b1, amat2, bmat2, conv2_root, bias2, pw2,
                           fc1t, fc1b, bn4g, bn4b, fc2t, fc2b, bn5g, bn5b,
                           fc3t, fc3b)]
    out_specs = [
        pl.BlockSpec((BPP, 1, 2), lambda i: (i, 0, 0)),
        pl.BlockSpec((BPP, 1, k1), lambda i: (i, 0, 0)),
        pl.BlockSpec((BPP, 1, k2), lambda i: (i, 0, 0)),
    ]
    out_shape = [
        jax.ShapeDtypeStruct((G, 1, 2), f32),
        jax.ShapeDtypeStruct((G, 1, k1), f32),
        jax.ShapeDtypeStruct((G, 1, k2), f32),
    ]
    out, sc1, sc2 = pl.pallas_call(
        _li_net_kernel,
        grid=grid,
        in_specs=in_specs,
        out_specs=out_specs,
        out_shape=out_shape,
    )(s, d, w, xg, n1w1, n1b1, amat1, bmat1, conv1_root, bias1, pw1,
      n2w1, n2b1, amat2, bmat2, conv2_root, bias2, pw2,
      fc1t, fc1b, bn4g, bn4b, fc2t, fc2b, bn5g, bn5b, fc3t, fc3b)
    return (out.reshape(G, 2), sc1.reshape(-1), sc2.reshape(-1))


# speedup vs baseline: 2.9517x; 1.0131x over previous
import jax
import jax.numpy as jnp
from jax.experimental import pallas as pl

G = 256
NPG = 128
INDIM = 128
EPG = 2048
D1 = 32
D2 = 32
D3 = 8
BN_EPS = 1e-5
ECH = 128
NCH = EPG // ECH

f32 = jnp.float32
bf16 = jnp.bfloat16


def _bdot(a, b, dims):
    return jax.lax.dot_general(a.astype(bf16), b.astype(bf16), (dims, ((), ())),
                               preferred_element_type=f32)


def _fdot(a, b, dims):
    return jax.lax.dot_general(a, b, (dims, ((), ())),
                               preferred_element_type=f32,
                               precision=jax.lax.Precision.HIGHEST)


def _augment(A, n):
    ii = jax.lax.broadcasted_iota(jnp.int32, (n, n), 0)
    jj = jax.lax.broadcasted_iota(jnp.int32, (n, n), 1)
    eye = (ii == jj).astype(f32)
    Ai = A + eye
    A2 = _bdot(Ai, Ai, ((1,), (0,)))
    return A2 * (1.0 - eye)


def _nnconv(xg, A2, w1, b1, Amat, Bmat, root, bias, precise_root=False):
    M = (A2 != 0.0).astype(f32)
    H = jax.nn.relu(A2 * w1 + b1) * M
    XA = _bdot(xg, Amat, ((1,), (0,)))
    XB = _bdot(xg, Bmat, ((1,), (0,)))
    aggr = _bdot(M, XB, ((0,), (0,)))
    aggr = aggr + _bdot(H, XA, ((0,), (0,)))
    dot = _fdot if precise_root else _bdot
    XR = dot(xg, root, ((1,), (0,)))
    return aggr + XR + bias


def _topk_perm(h, pw_col, n, k, precise_score=False):
    nrm = jnp.sqrt(jnp.sum(pw_col * pw_col))
    dot = _fdot if precise_score else _bdot
    sc_col = jnp.tanh(dot(h, pw_col, ((1,), (0,))) / nrm)
    sc_row = sc_col.reshape(1, n)
    ii = jax.lax.broadcasted_iota(jnp.int32, (n, n), 0)
    jj = jax.lax.broadcasted_iota(jnp.int32, (n, n), 1)
    gt = sc_col > sc_row
    eq = jnp.logical_and(sc_col == sc_row, ii < jj)
    Gm = jnp.logical_or(gt, eq).astype(bf16)
    rank_row = _bdot(jnp.ones((1, n), f32), Gm, ((1,), (0,)))
    rr = jax.lax.broadcasted_iota(jnp.int32, (k, n), 0).astype(f32)
    P = (rank_row == rr).astype(f32)
    sel_col = _fdot(P, sc_col, ((1,), (0,)))
    return P, sel_col


BPP = 8


def _li_net_kernel(s_ref, d_ref, w_ref, xg_ref,
                   n1w1_ref, n1b1_ref, amat1_ref, bmat1_ref, root1_ref,
                   bias1_ref, pw1_ref,
                   n2w1_ref, n2b1_ref, amat2_ref, bmat2_ref, root2_ref,
                   bias2_ref, pw2_ref,
                   fc1_ref, fc1b_ref, bn4g_ref, bn4b_ref,
                   fc2_ref, fc2b_ref, bn5g_ref, bn5b_ref,
                   fc3_ref, fc3b_ref,
                   out_ref, sc1_ref, sc2_ref):
    k1 = NPG // 2
    k2 = k1 // 2
    for g in range(BPP):
        srow = s_ref[g]
        drow = d_ref[g]
        wrow = w_ref[g]
        xg = xg_ref[g]

        iota_n = jax.lax.broadcasted_iota(jnp.int32, (NPG, EPG), 0)
        sT = (iota_n == srow).astype(f32)
        dT = (iota_n == drow).astype(f32)
        sTb = sT.astype(bf16)
        dTb = dT.astype(bf16)
        q0 = wrow.astype(bf16).astype(f32)
        q1 = (wrow - q0).astype(bf16).astype(f32)
        q2 = wrow - q0 - q1
        A = jnp.zeros((NPG, NPG), f32)
        for piece in (q0, q1, q2):
            dw = dTb * piece.astype(bf16)
            A = A + jax.lax.dot_general(sTb, dw, ((((1,), (1,))), ((), ())),
                                        preferred_element_type=f32)

        A2 = _augment(A, NPG)
        h1 = _nnconv(xg, A2, n1w1_ref[:], n1b1_ref[:], amat1_ref[:],
                     bmat1_ref[:], root1_ref[:], bias1_ref[:])
        P1, sel1 = _topk_perm(h1, pw1_ref[:], NPG, k1)
        xk1 = _fdot(P1, h1, ((1,), (0,))) * sel1
        AP = _fdot(A2, P1, ((1,), (1,)))
        A1p = _fdot(P1, AP, ((1,), (0,)))
        x1 = jnp.concatenate([jnp.max(xk1, axis=0, keepdims=True),
                              jnp.mean(xk1, axis=0, keepdims=True)], axis=1)

        A2b = _augment(A1p, k1)
        h2 = _nnconv(xk1, A2b, n2w1_ref[:], n2b1_ref[:], amat2_ref[:],
                     bmat2_ref[:], root2_ref[:], bias2_ref[:],
                     precise_root=True)
        P2, sel2 = _topk_perm(h2, pw2_ref[:], k1, k2, precise_score=True)
        xk2 = _fdot(P2, h2, ((1,), (0,))) * sel2
        x2 = jnp.concatenate([jnp.max(xk2, axis=0, keepdims=True),
                              jnp.mean(xk2, axis=0, keepdims=True)], axis=1)

        xcat = jnp.concatenate([x1, x2], axis=1)
        rsq = jnp.sqrt(1.0 + BN_EPS)
        h = _bdot(xcat, fc1_ref[:], ((1,), (0,))) + fc1b_ref[:]
        h = jax.nn.relu(bn4g_ref[:] * h / rsq + bn4b_ref[:])
        h = _bdot(h, fc2_ref[:], ((1,), (0,))) + fc2b_ref[:]
        h = jax.nn.relu(bn5g_ref[:] * h / rsq + bn5b_ref[:])
        lg = _bdot(h, fc3_ref[:], ((1,), (0,))) + fc3b_ref[:]
        m = jnp.max(lg, axis=1, keepdims=True)
        lse = m + jnp.log(jnp.sum(jnp.exp(lg - m), axis=1, keepdims=True))
        out_ref[g] = lg - lse
        sc1_ref[g] = sel1.reshape(1, k1)
        sc2_ref[g] = sel2.reshape(1, k2)


def kernel(x, edge_index, batch, edge_attr, n1_w1, n1_b1, n1_w2, n1_b2,
           conv1_root, conv1_bias, pool1_w, n2_w1, n2_b1, n2_w2, n2_b2,
           conv2_root, conv2_bias, pool2_w, fc1_w, fc1_b, bn4_g, bn4_b,
           fc2_w, fc2_b, bn5_g, bn5_b, fc3_w, fc3_b):
    k1 = NPG // 2
    k2 = k1 // 2
    s = (edge_index[0] % NPG).astype(jnp.int32).reshape(G, 1, EPG)
    d = (edge_index[1] % NPG).astype(jnp.int32).reshape(G, 1, EPG)
    w = edge_attr.reshape(G, 1, EPG).astype(f32)
    xg = x.reshape(G, NPG, INDIM)

    n1w1 = n1_w1.reshape(1, 1)
    n1b1 = n1_b1.reshape(1, 1)
    amat1 = n1_w2.reshape(INDIM, D1)
    bmat1 = n1_b2.reshape(INDIM, D1)
    bias1 = conv1_bias.reshape(1, D1)
    pw1 = pool1_w.reshape(D1, 1)
    n2w1 = n2_w1.reshape(1, 1)
    n2b1 = n2_b1.reshape(1, 1)
    amat2 = n2_w2.reshape(D1, D2)
    bmat2 = n2_b2.reshape(D1, D2)
    bias2 = conv2_bias.reshape(1, D2)
    pw2 = pool2_w.reshape(D2, 1)
    fc1t = fc1_w.T
    fc2t = fc2_w.T
    fc3t = fc3_w.T
    fc1b = fc1_b.reshape(1, D2)
    fc2b = fc2_b.reshape(1, D3)
    fc3b = fc3_b.reshape(1, 2)
    bn4g = bn4_g.reshape(1, D2)
    bn4b = bn4_b.reshape(1, D2)
    bn5g = bn5_g.reshape(1, D3)
    bn5b = bn5_b.reshape(1, D3)

    def full(a):
        return pl.BlockSpec(a.shape, lambda i: (0,) * a.ndim)

    grid = (G // BPP,)
    in_specs = [
        pl.BlockSpec((BPP, 1, EPG), lambda i: (i, 0, 0)),
        pl.BlockSpec((BPP, 1, EPG), lambda i: (i, 0, 0)),
        pl.BlockSpec((BPP, 1, EPG), lambda i: (i, 0, 0)),
        pl.BlockSpec((BPP, NPG, INDIM), lambda i: (i, 0, 0)),
    ] + [full(a) for a in (n1w1, n1b1, amat1, bmat1, conv1_root, bias1, pw1,
                           n2w1, n2b1, amat2, bmat2, conv2_root, bias2, pw2,
                           fc1t, fc1b, bn4g, bn4b, fc2t, fc2b, bn5g, bn5b,
                           fc3t, fc3b)]
    out_specs = [
        pl.BlockSpec((BPP, 1, 2), lambda i: (i, 0, 0)),
        pl.BlockSpec((BPP, 1, k1), lambda i: (i, 0, 0)),
        pl.BlockSpec((BPP, 1, k2), lambda i: (i, 0, 0)),
    ]
    out_shape = [
        jax.ShapeDtypeStruct((G, 1, 2), f32),
        jax.ShapeDtypeStruct((G, 1, k1), f32),
        jax.ShapeDtypeStruct((G, 1, k2), f32),
    ]
    out, sc1, sc2 = pl.pallas_call(
        _li_net_kernel,
        grid=grid,
        in_specs=in_specs,
        out_specs=out_specs,
        out_shape=out_shape,
    )(s, d, w, xg, n1w1, n1b1, amat1, bmat1, conv1_root, bias1, pw1,
      n2w1, n2b1, amat2, bmat2, conv2_root, bias2, pw2,
      fc1t, fc1b, bn4g, bn4b, fc2t, fc2b, bn5g, bn5b, fc3t, fc3b)
    return (out.reshape(G, 2), sc1.reshape(-1), sc2.reshape(-1))


# speedup vs baseline: 2.9712x; 1.0066x over previous
import jax
import jax.numpy as jnp
from jax.experimental import pallas as pl

G = 256
NPG = 128
INDIM = 128
EPG = 2048
D1 = 32
D2 = 32
D3 = 8
BN_EPS = 1e-5
ECH = 128
NCH = EPG // ECH

f32 = jnp.float32
bf16 = jnp.bfloat16


def _bdot(a, b, dims):
    return jax.lax.dot_general(a.astype(bf16), b.astype(bf16), (dims, ((), ())),
                               preferred_element_type=f32)


def _fdot(a, b, dims):
    return jax.lax.dot_general(a, b, (dims, ((), ())),
                               preferred_element_type=f32,
                               precision=jax.lax.Precision.HIGHEST)


def _augment(A, n):
    ii = jax.lax.broadcasted_iota(jnp.int32, (n, n), 0)
    jj = jax.lax.broadcasted_iota(jnp.int32, (n, n), 1)
    eye = (ii == jj).astype(f32)
    Ai = A + eye
    A2 = _bdot(Ai, Ai, ((1,), (0,)))
    return A2 * (1.0 - eye)


def _nnconv(xg, A2, w1, b1, Amat, Bmat, root, bias, precise_root=False):
    M = (A2 != 0.0).astype(f32)
    H = jax.nn.relu(A2 * w1 + b1) * M
    XA = _bdot(xg, Amat, ((1,), (0,)))
    XB = _bdot(xg, Bmat, ((1,), (0,)))
    aggr = _bdot(M, XB, ((0,), (0,)))
    aggr = aggr + _bdot(H, XA, ((0,), (0,)))
    dot = _fdot if precise_root else _bdot
    XR = dot(xg, root, ((1,), (0,)))
    return aggr + XR + bias


def _topk_perm(h, pw_col, n, k, precise_score=False):
    nrm = jnp.sqrt(jnp.sum(pw_col * pw_col))
    dot = _fdot if precise_score else _bdot
    sc_col = jnp.tanh(dot(h, pw_col, ((1,), (0,))) / nrm)
    sc_row = sc_col.reshape(1, n)
    ii = jax.lax.broadcasted_iota(jnp.int32, (n, n), 0)
    jj = jax.lax.broadcasted_iota(jnp.int32, (n, n), 1)
    gt = sc_col > sc_row
    eq = jnp.logical_and(sc_col == sc_row, ii < jj)
    Gm = jnp.logical_or(gt, eq).astype(bf16)
    rank_row = _bdot(jnp.ones((1, n), f32), Gm, ((1,), (0,)))
    rr = jax.lax.broadcasted_iota(jnp.int32, (k, n), 0).astype(f32)
    P = (rank_row == rr).astype(f32)
    sel_col = _fdot(P, sc_col, ((1,), (0,)))
    return P, sel_col


BPP = 16


def _li_net_kernel(s_ref, d_ref, w_ref, xg_ref,
                   n1w1_ref, n1b1_ref, amat1_ref, bmat1_ref, root1_ref,
                   bias1_ref, pw1_ref,
                   n2w1_ref, n2b1_ref, amat2_ref, bmat2_ref, root2_ref,
                   bias2_ref, pw2_ref,
                   fc1_ref, fc1b_ref, bn4g_ref, bn4b_ref,
                   fc2_ref, fc2b_ref, bn5g_ref, bn5b_ref,
                   fc3_ref, fc3b_ref,
                   out_ref, sc1_ref, sc2_ref):
    k1 = NPG // 2
    k2 = k1 // 2
    for g in range(BPP):
        srow = s_ref[g]
        drow = d_ref[g]
        wrow = w_ref[g]
        xg = xg_ref[g]

        iota_n = jax.lax.broadcasted_iota(jnp.int32, (NPG, EPG), 0)
        sT = (iota_n == srow).astype(f32)
        dT = (iota_n == drow).astype(f32)
        sTb = sT.astype(bf16)
        dTb = dT.astype(bf16)
        q0 = wrow.astype(bf16).astype(f32)
        q1 = (wrow - q0).astype(bf16).astype(f32)
        q2 = wrow - q0 - q1
        A = jnp.zeros((NPG, NPG), f32)
        for piece in (q0, q1, q2):
            dw = dTb * piece.astype(bf16)
            A = A + jax.lax.dot_general(sTb, dw, ((((1,), (1,))), ((), ())),
                                        preferred_element_type=f32)

        A2 = _augment(A, NPG)
        h1 = _nnconv(xg, A2, n1w1_ref[:], n1b1_ref[:], amat1_ref[:],
                     bmat1_ref[:], root1_ref[:], bias1_ref[:])
        P1, sel1 = _topk_perm(h1, pw1_ref[:], NPG, k1)
        xk1 = _fdot(P1, h1, ((1,), (0,))) * sel1
        AP = _fdot(A2, P1, ((1,), (1,)))
        A1p = _fdot(P1, AP, ((1,), (0,)))
        x1 = jnp.concatenate([jnp.max(xk1, axis=0, keepdims=True),
                              jnp.mean(xk1, axis=0, keepdims=True)], axis=1)

        A2b = _augment(A1p, k1)
        h2 = _nnconv(xk1, A2b, n2w1_ref[:], n2b1_ref[:], amat2_ref[:],
                     bmat2_ref[:], root2_ref[:], bias2_ref[:],
                     precise_root=True)
        P2, sel2 = _topk_perm(h2, pw2_ref[:], k1, k2, precise_score=True)
        xk2 = _fdot(P2, h2, ((1,), (0,))) * sel2
        x2 = jnp.concatenate([jnp.max(xk2, axis=0, keepdims=True),
                              jnp.mean(xk2, axis=0, keepdims=True)], axis=1)

        xcat = jnp.concatenate([x1, x2], axis=1)
        rsq = jnp.sqrt(1.0 + BN_EPS)
        h = _bdot(xcat, fc1_ref[:], ((1,), (0,))) + fc1b_ref[:]
        h = jax.nn.relu(bn4g_ref[:] * h / rsq + bn4b_ref[:])
        h = _bdot(h, fc2_ref[:], ((1,), (0,))) + fc2b_ref[:]
        h = jax.nn.relu(bn5g_ref[:] * h / rsq + bn5b_ref[:])
        lg = _bdot(h, fc3_ref[:], ((1,), (0,))) + fc3b_ref[:]
        m = jnp.max(lg, axis=1, keepdims=True)
        lse = m + jnp.log(jnp.sum(jnp.exp(lg - m), axis=1, keepdims=True))
        out_ref[g] = lg - lse
        sc1_ref[g] = sel1.reshape(1, k1)
        sc2_ref[g] = sel2.reshape(1, k2)


def kernel(x, edge_index, batch, edge_attr, n1_w1, n1_b1, n1_w2, n1_b2,
           conv1_root, conv1_bias, pool1_w, n2_w1, n2_b1, n2_w2, n2_b2,
           conv2_root, conv2_bias, pool2_w, fc1_w, fc1_b, bn4_g, bn4_b,
           fc2_w, fc2_b, bn5_g, bn5_b, fc3_w, fc3_b):
    k1 = NPG // 2
    k2 = k1 // 2
    s = (edge_index[0] % NPG).astype(jnp.int32).reshape(G, 1, EPG)
    d = (edge_index[1] % NPG).astype(jnp.int32).reshape(G, 1, EPG)
    w = edge_attr.reshape(G, 1, EPG).astype(f32)
    xg = x.reshape(G, NPG, INDIM)

    n1w1 = n1_w1.reshape(1, 1)
    n1b1 = n1_b1.reshape(1, 1)
    amat1 = n1_w2.reshape(INDIM, D1)
    bmat1 = n1_b2.reshape(INDIM, D1)
    bias1 = conv1_bias.reshape(1, D1)
    pw1 = pool1_w.reshape(D1, 1)
    n2w1 = n2_w1.reshape(1, 1)
    n2b1 = n2_b1.reshape(1, 1)
    amat2 = n2_w2.reshape(D1, D2)
    bmat2 = n2_b2.reshape(D1, D2)
    bias2 = conv2_bias.reshape(1, D2)
    pw2 = pool2_w.reshape(D2, 1)
    fc1t = fc1_w.T
    fc2t = fc2_w.T
    fc3t = fc3_w.T
    fc1b = fc1_b.reshape(1, D2)
    fc2b = fc2_b.reshape(1, D3)
    fc3b = fc3_b.reshape(1, 2)
    bn4g = bn4_g.reshape(1, D2)
    bn4b = bn4_b.reshape(1, D2)
    bn5g = bn5_g.reshape(1, D3)
    bn5b = bn5_b.reshape(1, D3)

    def full(a):
        return pl.BlockSpec(a.shape, lambda i: (0,) * a.ndim)

    grid = (G // BPP,)
    in_specs = [
        pl.BlockSpec((BPP, 1, EPG), lambda i: (i, 0, 0)),
        pl.BlockSpec((BPP, 1, EPG), lambda i: (i, 0, 0)),
        pl.BlockSpec((BPP, 1, EPG), lambda i: (i, 0, 0)),
        pl.BlockSpec((BPP, NPG, INDIM), lambda i: (i, 0, 0)),
    ] + [full(a) for a in (n1w1, n1b1, amat1, bmat1, conv1_root, bias1, pw1,
                           n2w1, n2b1, amat2, bmat2, conv2_root, bias2, pw2,
                           fc1t, fc1b, bn4g, bn4b, fc2t, fc2b, bn5g, bn5b,
                           fc3t, fc3b)]
    out_specs = [
        pl.BlockSpec((BPP, 1, 2), lambda i: (i, 0, 0)),
        pl.BlockSpec((BPP, 1, k1), lambda i: (i, 0, 0)),
        pl.BlockSpec((BPP, 1, k2), lambda i: (i, 0, 0)),
    ]
    out_shape = [
        jax.ShapeDtypeStruct((G, 1, 2), f32),
        jax.ShapeDtypeStruct((G, 1, k1), f32),
        jax.ShapeDtypeStruct((G, 1, k2), f32),
    ]
    out, sc1, sc2 = pl.pallas_call(
        _li_net_kernel,
        grid=grid,
        in_specs=in_specs,
        out_specs=out_specs,
        out_shape=out_shape,
    )(s, d, w, xg, n1w1, n1b1, amat1, bmat1, conv1_root, bias1, pw1,
      n2w1, n2b1, amat2, bmat2, conv2_root, bias2, pw2,
      fc1t, fc1b, bn4g, bn4b, fc2t, fc2b, bn5g, bn5b, fc3t, fc3b)
    return (out.reshape(G, 2), sc1.reshape(-1), sc2.reshape(-1))
